# baseline (speedup 1.0000x reference)
import functools
import math

import jax
import jax.numpy as jnp
from jax import lax
from jax.experimental import pallas as pl
from jax.experimental.pallas import tpu as pltpu
from jax.experimental.pallas import tpu_sc as plsc

N = 10000
E = 320000
D_FEAT = 128
D_HID = 128
D_BERT = 768
B = 1024
L_ENT = 8
HEADS = 4
DH = D_HID // HEADS
COMB = D_BERT + D_HID

NP_ = 10240
CH = 128
NCH = 80
NPH = 5
PCH = NCH // NPH
EP = 32 * NCH * CH
ROWS_T = NP_ // 16

BB = 128
NBLK = 8
BN = NP_ // NBLK

_MESH = plsc.VectorSubcoreMesh(core_axis_name="c", subcore_axis_name="s")
_f32 = jnp.float32


@functools.partial(
    pl.kernel,
    out_type=jax.ShapeDtypeStruct((2, NP_, D_HID), _f32),
    mesh=_MESH,
    scratch_types=[
        pltpu.VMEM((PCH, CH), jnp.int32),
        pltpu.VMEM((CH, D_HID), _f32),
        pltpu.VMEM_SHARED((NP_, D_HID), _f32),
    ],
)
def _sc_degree(dst_hbm, ones_hbm, zeros_hbm, out_hbm, idx_v, ones_v, accd):
    c = lax.axis_index("c")
    s = lax.axis_index("s")
    pltpu.sync_copy(ones_hbm, ones_v)

    @pl.when(s == 0)
    def _():
        pltpu.sync_copy(zeros_hbm, accd)

    plsc.subcore_barrier()

    def phase(p, carry):
        pltpu.sync_copy(dst_hbm.at[c, s, pl.ds(p * PCH, PCH)], idx_v)

        def body(j, carry2):
            pltpu.sync_copy(ones_v, accd.at[idx_v.at[j]], add=True)
            return carry2

        return lax.fori_loop(0, PCH, body, carry)

    lax.fori_loop(0, NPH, phase, 0)
    plsc.subcore_barrier()
    pltpu.sync_copy(accd.at[pl.ds(s * ROWS_T, ROWS_T)],
                    out_hbm.at[c, pl.ds(s * ROWS_T, ROWS_T)])


@functools.partial(
    pl.kernel,
    out_type=jax.ShapeDtypeStruct((2, NP_, D_HID), _f32),
    mesh=_MESH,
    scratch_types=[
        pltpu.VMEM((NCH, CH), jnp.int32),
        pltpu.VMEM((PCH, CH), jnp.int32),
        pltpu.VMEM((CH, D_HID), _f32),
        pltpu.VMEM((CH, D_HID), _f32),
        pltpu.VMEM_SHARED((NP_, D_HID), _f32),
        pltpu.SemaphoreType.DMA,
        pltpu.SemaphoreType.DMA,
        pltpu.SemaphoreType.DMA,
        pltpu.SemaphoreType.DMA,
    ],
)
def _sc_scatter(y_hbm, src_hbm, dst_hbm, out_hbm,
                src_v, dst_v, buf_a, buf_b, acc, ga, gb, sa, sb):
    c = lax.axis_index("c")
    s = lax.axis_index("s")
    pltpu.sync_copy(src_hbm.at[c, s], src_v)

    @pl.when(s == 0)
    def _():
        pltpu.sync_copy(y_hbm, acc)

    plsc.subcore_barrier()
    pltpu.sync_copy(dst_hbm.at[c, s, pl.ds(0, PCH)], dst_v)
    pltpu.async_copy(y_hbm.at[src_v.at[0]], buf_a, ga)

    npair = NCH // 2
    hpch = PCH // 2

    def body(jp, carry):
        j0 = jp * 2
        j1 = j0 + 1
        pltpu.make_async_copy(y_hbm.at[src_v.at[j0]], buf_a, ga).wait()

        @pl.when(jp > 0)
        def _():
            pltpu.make_async_copy(buf_b, acc.at[dst_v.at[0]], sb).wait()

        @pl.when(lax.rem(jp, hpch) == 0)
        def _():
            pltpu.sync_copy(dst_hbm.at[c, s, pl.ds((jp // hpch) * PCH, PCH)],
                            dst_v)

        r0 = lax.rem(j0, PCH)
        r1 = lax.rem(j1, PCH)
        pltpu.async_copy(buf_a, acc.at[dst_v.at[r0]], sa, add=True)
        pltpu.async_copy(y_hbm.at[src_v.at[j1]], buf_b, gb)
        pltpu.make_async_copy(y_hbm.at[src_v.at[j1]], buf_b, gb).wait()
        pltpu.make_async_copy(buf_a, acc.at[dst_v.at[r0]], sa).wait()
        pltpu.async_copy(buf_b, acc.at[dst_v.at[r1]], sb, add=True)

        @pl.when(jp < npair - 1)
        def _():
            pltpu.async_copy(y_hbm.at[src_v.at[j0 + 2]], buf_a, ga)

        return carry

    lax.fori_loop(0, npair, body, 0)
    pltpu.make_async_copy(buf_b, acc.at[dst_v.at[0]], sb).wait()
    plsc.subcore_barrier()
    pltpu.sync_copy(acc.at[pl.ds(s * ROWS_T, ROWS_T)],
                    out_hbm.at[c, pl.ds(s * ROWS_T, ROWS_T)])


_GCH = (B * L_ENT) // 32 // CH


@functools.partial(
    pl.kernel,
    out_type=jax.ShapeDtypeStruct((B * L_ENT, D_HID), _f32),
    mesh=_MESH,
    scratch_types=[
        pltpu.VMEM((_GCH, CH), jnp.int32),
        pltpu.VMEM((CH, D_HID), _f32),
        pltpu.SemaphoreType.DMA,
    ],
)
def _sc_entity_gather(h_hbm, idx_hbm, out_hbm, idx_v, buf, sem):
    c = lax.axis_index("c")
    s = lax.axis_index("s")
    pltpu.sync_copy(idx_hbm.at[c, s], idx_v)
    wid = c * 16 + s
    for t in range(_GCH):
        pltpu.async_copy(h_hbm.at[idx_v.at[t]], buf, sem).wait()
        pltpu.sync_copy(buf, out_hbm.at[pl.ds((wid * _GCH + t) * CH, CH)])


def _xw_body(x_ref, w_ref, y_ref):
    y_ref[...] = lax.dot_general(x_ref[...], w_ref[...],
                                 (((1,), (0,)), ((), ())),
                                 preferred_element_type=_f32)


def _scale_body(xw_ref, deg_ref, y_ref):
    deg = deg_ref[0, :, 0] + deg_ref[1, :, 0] + 1.0
    dinv = lax.rsqrt(deg)
    y_ref[...] = xw_ref[...] * dinv[:, None]


def _y2_body(s_ref, y1_ref, deg_ref, w_ref, b_ref, y2_ref):
    deg = deg_ref[0, :, 0] + deg_ref[1, :, 0] + 1.0
    dinv = lax.rsqrt(deg)
    agg = (s_ref[0] + s_ref[1] - y1_ref[...]) * dinv[:, None] + b_ref[...][None, :]
    h1 = jnp.maximum(agg, 0.0)
    hw = lax.dot_general(h1, w_ref[...], (((1,), (0,)), ((), ())),
                         preferred_element_type=_f32)
    y2_ref[...] = hw * dinv[:, None]


def _h2_body(s_ref, y2_ref, deg_ref, b_ref, h2_ref):
    deg = deg_ref[0, :, 0] + deg_ref[1, :, 0] + 1.0
    dinv = lax.rsqrt(deg)
    agg = (s_ref[0] + s_ref[1] - y2_ref[...]) * dinv[:, None] + b_ref[...][None, :]
    h2_ref[...] = jnp.maximum(agg, 0.0)


_NODE_BLK = pl.BlockSpec((BN, D_HID), lambda b: (b, 0))
_PAIR_BLK = pl.BlockSpec((2, BN, D_HID), lambda b: (0, b, 0))
_DEG_BLK = pl.BlockSpec((2, BN, D_HID), lambda b: (0, b, 0))
_W_FULL = pl.BlockSpec((D_HID, D_HID), lambda b: (0, 0))
_B_FULL = pl.BlockSpec((D_HID,), lambda b: (0,))
_NODE_OUT = jax.ShapeDtypeStruct((NP_, D_HID), _f32)


def _tc_xw(xp, W1):
    return pl.pallas_call(
        _xw_body, grid=(NBLK,),
        in_specs=[_NODE_BLK, _W_FULL],
        out_specs=_NODE_BLK, out_shape=_NODE_OUT,
    )(xp, W1)


def _tc_scale(xw, degp):
    return pl.pallas_call(
        _scale_body, grid=(NBLK,),
        in_specs=[_NODE_BLK, _DEG_BLK],
        out_specs=_NODE_BLK, out_shape=_NODE_OUT,
    )(xw, degp)


def _tc_y2(S1, y1, degp, W2, b1):
    return pl.pallas_call(
        _y2_body, grid=(NBLK,),
        in_specs=[_PAIR_BLK, _NODE_BLK, _DEG_BLK, _W_FULL, _B_FULL],
        out_specs=_NODE_BLK, out_shape=_NODE_OUT,
    )(S1, y1, degp, W2, b1)


def _tc_h2(S2, y2, degp, b2):
    return pl.pallas_call(
        _h2_body, grid=(NBLK,),
        in_specs=[_PAIR_BLK, _NODE_BLK, _DEG_BLK, _B_FULL],
        out_specs=_NODE_BLK, out_shape=_NODE_OUT,
    )(S2, y2, degp, b2)


def _head_body(ent_ref, bert_ref, win_ref, binw_ref, wout_ref, bout_ref,
               bng_b_ref, bnb_b_ref, bng_g_ref, bnb_g_ref,
               fc1w_ref, fc1b_ref, fc2w_ref, fc2b_ref, fc3w_ref, fc3b_ref,
               out_ref):
    ent = ent_ref[...]
    ent2 = ent.reshape(BB * L_ENT, D_HID)
    qkv = lax.dot_general(ent2, win_ref[...],
                          (((1,), (1,)), ((), ())),
                          preferred_element_type=_f32)
    qkv = qkv + binw_ref[...][None, :]
    qkv3 = qkv.reshape(BB, L_ENT, 3 * D_HID)
    q = qkv3[:, :, :D_HID]
    k = qkv3[:, :, D_HID:2 * D_HID]
    v = qkv3[:, :, 2 * D_HID:]

    scale = 1.0 / math.sqrt(DH)
    s = [[None] * L_ENT for _ in range(L_ENT)]
    for i in range(L_ENT):
        qi = q[:, i, :]
        for j in range(L_ENT):
            p = qi * k[:, j, :]
            s[i][j] = p.reshape(BB, HEADS, DH).sum(axis=-1) * scale
    cs = []
    for i in range(L_ENT):
        m = s[i][0]
        for j in range(1, L_ENT):
            m = jnp.maximum(m, s[i][j])
        es = [jnp.exp(s[i][j] - m) for j in range(L_ENT)]
        z = es[0]
        for j in range(1, L_ENT):
            z = z + es[j]
        zi = 1.0 / z
        cs.append([e * zi for e in es])
    inv_l = 1.0 / L_ENT
    obar = None
    for j in range(L_ENT):
        cj = cs[0][j]
        for i in range(1, L_ENT):
            cj = cj + cs[i][j]
        cj = cj * inv_l
        cjb = jnp.broadcast_to(cj[:, :, None], (BB, HEADS, DH)).reshape(BB, D_HID)
        term = cjb * v[:, j, :]
        obar = term if obar is None else obar + term
    gnn = lax.dot_general(obar, wout_ref[...],
                          (((1,), (1,)), ((), ())),
                          preferred_element_type=_f32)
    gnn = gnn + bout_ref[...][None, :]

    sbn = 1.0 / math.sqrt(1.0 + 1e-5)
    gnn = gnn * (sbn * bng_g_ref[...][None, :]) + bnb_g_ref[...][None, :]
    bert = bert_ref[...] * (sbn * bng_b_ref[...][None, :]) + bnb_b_ref[...][None, :]

    fc1w = fc1w_ref[...]
    x1 = lax.dot_general(bert, fc1w[:, :D_BERT],
                         (((1,), (1,)), ((), ())),
                         preferred_element_type=_f32)
    x1 = x1 + lax.dot_general(gnn, fc1w[:, D_BERT:],
                              (((1,), (1,)), ((), ())),
                              preferred_element_type=_f32)
    x1 = jnp.maximum(x1 + fc1b_ref[...][None, :], 0.0)
    x2 = lax.dot_general(x1, fc2w_ref[...],
                         (((1,), (1,)), ((), ())),
                         preferred_element_type=_f32)
    x2 = jnp.maximum(x2 + fc2b_ref[...][None, :], 0.0)
    x3 = jnp.sum(x2 * fc3w_ref[...], axis=1, keepdims=True) + fc3b_ref[0]
    out_ref[...] = jax.nn.sigmoid(x3)


def _head(ent, bert, Win, bin_w, Wout, bout, bng_b, bnb_b, bng_g, bnb_g,
          fc1_w, fc1_b, fc2_w, fc2_b, fc3_w, fc3_b):
    full = lambda shape: pl.BlockSpec(shape, lambda b: (0,) * len(shape))
    return pl.pallas_call(
        _head_body,
        grid=(B // BB,),
        in_specs=[
            pl.BlockSpec((BB, L_ENT, D_HID), lambda b: (b, 0, 0)),
            pl.BlockSpec((BB, D_BERT), lambda b: (b, 0)),
            full((3 * D_HID, D_HID)),
            full((3 * D_HID,)),
            full((D_HID, D_HID)),
            full((D_HID,)),
            full((D_BERT,)),
            full((D_BERT,)),
            full((D_HID,)),
            full((D_HID,)),
            full((COMB // 2, COMB)),
            full((COMB // 2,)),
            full((COMB // 4, COMB // 2)),
            full((COMB // 4,)),
            full((1, COMB // 4)),
            pl.BlockSpec(memory_space=pltpu.MemorySpace.SMEM),
        ],
        out_specs=pl.BlockSpec((BB, 1), lambda b: (b, 0)),
        out_shape=jax.ShapeDtypeStruct((B, 1), jnp.float32),
    )(ent, bert, Win, bin_w, Wout, bout, bng_b, bnb_b, bng_g, bnb_g,
      fc1_w, fc1_b, fc2_w, fc2_b, fc3_w, fc3_b)


def kernel(article_bert_embeddings, x, edge_index, article_entity_map_tensor,
           W1, b1, W2, b2, bn_bert_g, bn_bert_b, bn_gnn_g, bn_gnn_b,
           Win, bin_w, Wout, bout, fc1_w, fc1_b, fc2_w, fc2_b, fc3_w, fc3_b):
    xp = jnp.pad(x, ((0, NP_ - N), (0, 0)))
    pad = N + jnp.arange(EP - E, dtype=edge_index.dtype) % (NP_ - N)
    srcp = jnp.concatenate([edge_index[0], pad]).reshape(2, 16, NCH, CH)
    dstp = jnp.concatenate([edge_index[1], pad]).reshape(2, 16, NCH, CH)
    ones_rows = jnp.ones((CH, D_HID), _f32)
    zeros_tbl = jnp.zeros((NP_, D_HID), _f32)

    degp = _sc_degree(dstp, ones_rows, zeros_tbl)

    xw1 = _tc_xw(xp, W1)
    y1 = _tc_scale(xw1, degp)
    S1 = _sc_scatter(y1, srcp, dstp)
    y2 = _tc_y2(S1, y1, degp, W2, b1)
    S2 = _sc_scatter(y2, srcp, dstp)
    h2 = _tc_h2(S2, y2, degp, b2)

    eidx = article_entity_map_tensor.reshape(2, 16, _GCH, CH)
    ent = _sc_entity_gather(h2, eidx).reshape(B, L_ENT, D_HID)

    return _head(ent, article_bert_embeddings, Win, bin_w, Wout, bout,
                 bn_bert_g, bn_bert_b, bn_gnn_g, bn_gnn_b,
                 fc1_w, fc1_b, fc2_w, fc2_b, fc3_w, fc3_b)


# speedup vs baseline: 1.3742x; 1.3742x over previous
import functools
import math

import jax
import jax.numpy as jnp
from jax import lax
from jax.experimental import pallas as pl
from jax.experimental.pallas import tpu as pltpu
from jax.experimental.pallas import tpu_sc as plsc

N = 10000
E = 320000
D_FEAT = 128
D_HID = 128
D_BERT = 768
B = 1024
L_ENT = 8
HEADS = 4
DH = D_HID // HEADS
COMB = D_BERT + D_HID

NP_ = 10240
CH = 128
NCH = 80
NPH = 5
PCH = NCH // NPH
EP = 32 * NCH * CH
ROWS_T = NP_ // 16

BB = 128
NBLK = 8
BN = NP_ // NBLK

_MESH = plsc.VectorSubcoreMesh(core_axis_name="c", subcore_axis_name="s")
_f32 = jnp.float32


@functools.partial(
    pl.kernel,
    out_type=jax.ShapeDtypeStruct((2, NP_, D_HID), _f32),
    mesh=_MESH,
    scratch_types=[
        pltpu.VMEM((PCH, CH), jnp.int32),
        pltpu.VMEM((CH, D_HID), _f32),
        pltpu.VMEM_SHARED((NP_, D_HID), _f32),
    ],
)
def _sc_degree(dst_hbm, ones_hbm, zeros_hbm, out_hbm, idx_v, ones_v, accd):
    c = lax.axis_index("c")
    s = lax.axis_index("s")
    pltpu.sync_copy(ones_hbm, ones_v)

    @pl.when(s == 0)
    def _():
        pltpu.sync_copy(zeros_hbm, accd)

    plsc.subcore_barrier()

    def phase(p, carry):
        pltpu.sync_copy(dst_hbm.at[c, s, pl.ds(p * PCH, PCH)], idx_v)

        def body(j, carry2):
            pltpu.sync_copy(ones_v, accd.at[idx_v.at[j]], add=True)
            return carry2

        return lax.fori_loop(0, PCH, body, carry)

    lax.fori_loop(0, NPH, phase, 0)
    plsc.subcore_barrier()
    pltpu.sync_copy(accd.at[pl.ds(s * ROWS_T, ROWS_T)],
                    out_hbm.at[c, pl.ds(s * ROWS_T, ROWS_T)])


@functools.partial(
    pl.kernel,
    out_type=jax.ShapeDtypeStruct((2, NP_, D_HID), _f32),
    mesh=_MESH,
    scratch_types=[
        pltpu.VMEM((NCH, CH), jnp.int32),
        pltpu.VMEM((PCH, CH), jnp.int32),
        pltpu.VMEM((CH, D_HID), _f32),
        pltpu.VMEM((CH, D_HID), _f32),
        pltpu.VMEM_SHARED((NP_, D_HID), _f32),
        pltpu.SemaphoreType.DMA,
        pltpu.SemaphoreType.DMA,
        pltpu.SemaphoreType.DMA,
        pltpu.SemaphoreType.DMA,
    ],
)
def _sc_scatter(y_hbm, src_hbm, dst_hbm, out_hbm,
                src_v, dst_v, buf_a, buf_b, acc, ga, gb, sa, sb):
    c = lax.axis_index("c")
    s = lax.axis_index("s")
    pltpu.sync_copy(src_hbm.at[c, s], src_v)

    @pl.when(s == 0)
    def _():
        pltpu.sync_copy(y_hbm, acc)

    plsc.subcore_barrier()
    pltpu.sync_copy(dst_hbm.at[c, s, pl.ds(0, PCH)], dst_v)
    pltpu.async_copy(y_hbm.at[src_v.at[0]], buf_a, ga)

    npair = NCH // 2
    hpch = PCH // 2

    def body(jp, carry):
        j0 = jp * 2
        j1 = j0 + 1
        pltpu.make_async_copy(y_hbm.at[src_v.at[j0]], buf_a, ga).wait()

        @pl.when(jp > 0)
        def _():
            pltpu.make_async_copy(buf_b, acc.at[dst_v.at[0]], sb).wait()

        @pl.when(lax.rem(jp, hpch) == 0)
        def _():
            pltpu.sync_copy(dst_hbm.at[c, s, pl.ds((jp // hpch) * PCH, PCH)],
                            dst_v)

        r0 = lax.rem(j0, PCH)
        r1 = lax.rem(j1, PCH)
        pltpu.async_copy(buf_a, acc.at[dst_v.at[r0]], sa, add=True)
        pltpu.async_copy(y_hbm.at[src_v.at[j1]], buf_b, gb)
        pltpu.make_async_copy(y_hbm.at[src_v.at[j1]], buf_b, gb).wait()
        pltpu.make_async_copy(buf_a, acc.at[dst_v.at[r0]], sa).wait()
        pltpu.async_copy(buf_b, acc.at[dst_v.at[r1]], sb, add=True)

        @pl.when(jp < npair - 1)
        def _():
            pltpu.async_copy(y_hbm.at[src_v.at[j0 + 2]], buf_a, ga)

        return carry

    lax.fori_loop(0, npair, body, 0)
    pltpu.make_async_copy(buf_b, acc.at[dst_v.at[0]], sb).wait()
    plsc.subcore_barrier()
    pltpu.sync_copy(acc.at[pl.ds(s * ROWS_T, ROWS_T)],
                    out_hbm.at[c, pl.ds(s * ROWS_T, ROWS_T)])


_GCH = (B * L_ENT) // 32 // CH


@functools.partial(
    pl.kernel,
    out_type=jax.ShapeDtypeStruct((B * L_ENT, D_HID), _f32),
    mesh=_MESH,
    scratch_types=[
        pltpu.VMEM((_GCH, CH), jnp.int32),
        pltpu.VMEM((CH, D_HID), _f32),
        pltpu.SemaphoreType.DMA,
    ],
)
def _sc_entity_gather(h_hbm, idx_hbm, out_hbm, idx_v, buf, sem):
    c = lax.axis_index("c")
    s = lax.axis_index("s")
    pltpu.sync_copy(idx_hbm.at[c, s], idx_v)
    wid = c * 16 + s
    for t in range(_GCH):
        pltpu.async_copy(h_hbm.at[idx_v.at[t]], buf, sem).wait()
        pltpu.sync_copy(buf, out_hbm.at[pl.ds((wid * _GCH + t) * CH, CH)])


def _xw_body(x_ref, w_ref, y_ref):
    y_ref[...] = lax.dot_general(x_ref[...], w_ref[...],
                                 (((1,), (0,)), ((), ())),
                                 preferred_element_type=_f32)


def _scale_body(xw_ref, deg_ref, y_ref):
    deg = deg_ref[0, :, 0] + deg_ref[1, :, 0] + 1.0
    dinv = lax.rsqrt(deg)
    y_ref[...] = xw_ref[...] * dinv[:, None]


def _y2_body(s_ref, y1_ref, deg_ref, w_ref, b_ref, y2_ref):
    deg = deg_ref[0, :, 0] + deg_ref[1, :, 0] + 1.0
    dinv = lax.rsqrt(deg)
    agg = (s_ref[0] + s_ref[1] - y1_ref[...]) * dinv[:, None] + b_ref[...][None, :]
    h1 = jnp.maximum(agg, 0.0)
    hw = lax.dot_general(h1, w_ref[...], (((1,), (0,)), ((), ())),
                         preferred_element_type=_f32)
    y2_ref[...] = hw * dinv[:, None]


def _h2_body(s_ref, y2_ref, deg_ref, b_ref, h2_ref):
    deg = deg_ref[0, :, 0] + deg_ref[1, :, 0] + 1.0
    dinv = lax.rsqrt(deg)
    agg = (s_ref[0] + s_ref[1] - y2_ref[...]) * dinv[:, None] + b_ref[...][None, :]
    h2_ref[...] = jnp.maximum(agg, 0.0)


_NODE_BLK = pl.BlockSpec((BN, D_HID), lambda b: (b, 0))
_PAIR_BLK = pl.BlockSpec((2, BN, D_HID), lambda b: (0, b, 0))
_DEG_BLK = pl.BlockSpec((2, BN, D_HID), lambda b: (0, b, 0))
_W_FULL = pl.BlockSpec((D_HID, D_HID), lambda b: (0, 0))
_B_FULL = pl.BlockSpec((D_HID,), lambda b: (0,))
_NODE_OUT = jax.ShapeDtypeStruct((NP_, D_HID), _f32)


def _tc_xw(xp, W1):
    return pl.pallas_call(
        _xw_body, grid=(NBLK,),
        in_specs=[_NODE_BLK, _W_FULL],
        out_specs=_NODE_BLK, out_shape=_NODE_OUT,
    )(xp, W1)


def _tc_scale(xw, degp):
    return pl.pallas_call(
        _scale_body, grid=(NBLK,),
        in_specs=[_NODE_BLK, _DEG_BLK],
        out_specs=_NODE_BLK, out_shape=_NODE_OUT,
    )(xw, degp)


def _tc_y2(S1, y1, degp, W2, b1):
    return pl.pallas_call(
        _y2_body, grid=(NBLK,),
        in_specs=[_PAIR_BLK, _NODE_BLK, _DEG_BLK, _W_FULL, _B_FULL],
        out_specs=_NODE_BLK, out_shape=_NODE_OUT,
    )(S1, y1, degp, W2, b1)


def _tc_h2(S2, y2, degp, b2):
    return pl.pallas_call(
        _h2_body, grid=(NBLK,),
        in_specs=[_PAIR_BLK, _NODE_BLK, _DEG_BLK, _B_FULL],
        out_specs=_NODE_BLK, out_shape=_NODE_OUT,
    )(S2, y2, degp, b2)


def _head_body(ent_ref, bert_ref, win_ref, binw_ref, wout_ref, bout_ref,
               bng_b_ref, bnb_b_ref, bng_g_ref, bnb_g_ref,
               fc1w_ref, fc1b_ref, fc2w_ref, fc2b_ref, fc3w_ref, fc3b_ref,
               out_ref):
    ent = ent_ref[...]
    ent2 = ent.reshape(BB * L_ENT, D_HID)
    qkv = lax.dot_general(ent2, win_ref[...],
                          (((1,), (1,)), ((), ())),
                          preferred_element_type=_f32)
    qkv = qkv + binw_ref[...][None, :]
    qkv3 = qkv.reshape(BB, L_ENT, 3 * D_HID)
    q = qkv3[:, :, :D_HID]
    k = qkv3[:, :, D_HID:2 * D_HID]
    v = qkv3[:, :, 2 * D_HID:]

    scale = 1.0 / math.sqrt(DH)
    lanes = lax.broadcasted_iota(jnp.int32, (D_HID, HEADS), 0) // DH
    heads = lax.broadcasted_iota(jnp.int32, (D_HID, HEADS), 1)
    G = (lanes == heads).astype(_f32)

    csum = None
    for i in range(L_ENT):
        qi = q[:, i, :]
        prod = qi[:, None, :] * k
        si = lax.dot_general(prod.reshape(BB * L_ENT, D_HID), G,
                             (((1,), (0,)), ((), ())),
                             preferred_element_type=_f32) * scale
        si3 = si.reshape(BB, L_ENT, HEADS)
        m = jnp.max(si3, axis=1, keepdims=True)
        e = jnp.exp(si3 - m)
        z = jnp.sum(e, axis=1, keepdims=True)
        att_i = e / z
        csum = att_i if csum is None else csum + att_i
    c = csum * (1.0 / L_ENT)
    cb = lax.dot_general(c.reshape(BB * L_ENT, HEADS), G,
                         (((1,), (1,)), ((), ())),
                         preferred_element_type=_f32)
    obar = jnp.sum(cb.reshape(BB, L_ENT, D_HID) * v, axis=1)
    gnn = lax.dot_general(obar, wout_ref[...],
                          (((1,), (1,)), ((), ())),
                          preferred_element_type=_f32)
    gnn = gnn + bout_ref[...][None, :]

    sbn = 1.0 / math.sqrt(1.0 + 1e-5)
    gnn = gnn * (sbn * bng_g_ref[...][None, :]) + bnb_g_ref[...][None, :]
    bert = bert_ref[...] * (sbn * bng_b_ref[...][None, :]) + bnb_b_ref[...][None, :]

    fc1w = fc1w_ref[...]
    x1 = lax.dot_general(bert, fc1w[:, :D_BERT],
                         (((1,), (1,)), ((), ())),
                         preferred_element_type=_f32)
    x1 = x1 + lax.dot_general(gnn, fc1w[:, D_BERT:],
                              (((1,), (1,)), ((), ())),
                              preferred_element_type=_f32)
    x1 = jnp.maximum(x1 + fc1b_ref[...][None, :], 0.0)
    x2 = lax.dot_general(x1, fc2w_ref[...],
                         (((1,), (1,)), ((), ())),
                         preferred_element_type=_f32)
    x2 = jnp.maximum(x2 + fc2b_ref[...][None, :], 0.0)
    x3 = jnp.sum(x2 * fc3w_ref[...], axis=1, keepdims=True) + fc3b_ref[0]
    out_ref[...] = jax.nn.sigmoid(x3)


def _head(ent, bert, Win, bin_w, Wout, bout, bng_b, bnb_b, bng_g, bnb_g,
          fc1_w, fc1_b, fc2_w, fc2_b, fc3_w, fc3_b):
    full = lambda shape: pl.BlockSpec(shape, lambda b: (0,) * len(shape))
    return pl.pallas_call(
        _head_body,
        grid=(B // BB,),
        in_specs=[
            pl.BlockSpec((BB, L_ENT, D_HID), lambda b: (b, 0, 0)),
            pl.BlockSpec((BB, D_BERT), lambda b: (b, 0)),
            full((3 * D_HID, D_HID)),
            full((3 * D_HID,)),
            full((D_HID, D_HID)),
            full((D_HID,)),
            full((D_BERT,)),
            full((D_BERT,)),
            full((D_HID,)),
            full((D_HID,)),
            full((COMB // 2, COMB)),
            full((COMB // 2,)),
            full((COMB // 4, COMB // 2)),
            full((COMB // 4,)),
            full((1, COMB // 4)),
            pl.BlockSpec(memory_space=pltpu.MemorySpace.SMEM),
        ],
        out_specs=pl.BlockSpec((BB, 1), lambda b: (b, 0)),
        out_shape=jax.ShapeDtypeStruct((B, 1), jnp.float32),
    )(ent, bert, Win, bin_w, Wout, bout, bng_b, bnb_b, bng_g, bnb_g,
      fc1_w, fc1_b, fc2_w, fc2_b, fc3_w, fc3_b)


def kernel(article_bert_embeddings, x, edge_index, article_entity_map_tensor,
           W1, b1, W2, b2, bn_bert_g, bn_bert_b, bn_gnn_g, bn_gnn_b,
           Win, bin_w, Wout, bout, fc1_w, fc1_b, fc2_w, fc2_b, fc3_w, fc3_b):
    xp = jnp.pad(x, ((0, NP_ - N), (0, 0)))
    pad = N + jnp.arange(EP - E, dtype=edge_index.dtype) % (NP_ - N)
    srcp = jnp.concatenate([edge_index[0], pad]).reshape(2, 16, NCH, CH)
    dstp = jnp.concatenate([edge_index[1], pad]).reshape(2, 16, NCH, CH)
    ones_rows = jnp.ones((CH, D_HID), _f32)
    zeros_tbl = jnp.zeros((NP_, D_HID), _f32)

    degp = _sc_degree(dstp, ones_rows, zeros_tbl)

    xw1 = _tc_xw(xp, W1)
    y1 = _tc_scale(xw1, degp)
    S1 = _sc_scatter(y1, srcp, dstp)
    y2 = _tc_y2(S1, y1, degp, W2, b1)
    S2 = _sc_scatter(y2, srcp, dstp)
    h2 = _tc_h2(S2, y2, degp, b2)

    eidx = article_entity_map_tensor.reshape(2, 16, _GCH, CH)
    ent = _sc_entity_gather(h2, eidx).reshape(B, L_ENT, D_HID)

    return _head(ent, article_bert_embeddings, Win, bin_w, Wout, bout,
                 bn_bert_g, bn_bert_b, bn_gnn_g, bn_gnn_b,
                 fc1_w, fc1_b, fc2_w, fc2_b, fc3_w, fc3_b)
